# baseline (speedup 1.0000x reference)
import jax
import jax.numpy as jnp
from jax import lax
from jax.experimental import pallas as pl
from jax.experimental.pallas import tpu as pltpu
from jax.experimental.pallas import tpu_sc as plsc

_N = 10000
_E = 320000
_IN = 128
_H = 3
_F = 64
_HF = _H * _F
_K = 3
_NEG = 0.2

_NW = 32
_EPW = _E // _NW
_C = 80
_NCHUNK = _EPW // _C
_G = _C // 16
_NP = 10240
_RPT = _NP // 16
_NH = _NP // 2
_RPTH = _NH // 16
_EPT = _E // 16
_NCH2 = _EPT // _C
_ZR = 32

_BLK = 1000
_BLKP = 1024


def _leaky(x):
    return jnp.where(x >= 0, x, _NEG * x)


def _fc_body(feat_ref, wt_ref, a8_ref, ft_ref, tab1_ref):
    f = jnp.dot(feat_ref[...], wt_ref[...], preferred_element_type=jnp.float32)
    ft_ref[...] = f
    tab1_ref[...] = jnp.dot(f, a8_ref[...], preferred_element_type=jnp.float32)


def _fc(feat, wt, a8):
    return pl.pallas_call(
        _fc_body,
        grid=(_N // _BLK,),
        in_specs=[pl.BlockSpec((_BLK, _IN), lambda i: (i, 0)),
                  pl.BlockSpec((_IN, _HF), lambda i: (0, 0)),
                  pl.BlockSpec((_HF, 8), lambda i: (0, 0))],
        out_specs=[pl.BlockSpec((_BLK, _HF), lambda i: (i, 0)),
                   pl.BlockSpec((_BLK, 8), lambda i: (i, 0))],
        out_shape=[jax.ShapeDtypeStruct((_N, _HF), jnp.float32),
                   jax.ShapeDtypeStruct((_N, 8), jnp.float32)],
    )(feat, wt, a8)


def _edge1_body(ebuf_hbm, tab1_hbm, sacc_hbm,
                tab_v, sd_v, sidx_v, didx_v, ee_v, zbuf_v, acc_sh):
    c = lax.axis_index("c")
    s = lax.axis_index("s")
    wid = s * 2 + c
    pltpu.sync_copy(tab1_hbm, tab_v)

    z16 = jnp.zeros((16,), jnp.float32)
    lane = lax.iota(jnp.int32, 16)
    basis = [jnp.where(lane == k, 1.0, 0.0) for k in range(4)]

    def _zrow(i, carry):
        zbuf_v[i] = z16
        return carry
    lax.fori_loop(0, _RPT, _zrow, 0)
    pltpu.sync_copy(zbuf_v, acc_sh.at[pl.ds(s * _RPT, _RPT)])
    plsc.subcore_barrier()

    def _chunk(t, carry):
        pltpu.sync_copy(ebuf_hbm.at[wid * _NCHUNK + t], sd_v)
        for j in range(_C // 16):
            sidx_v[pl.ds(j * 16, 16)] = sd_v[pl.ds(j * 16, 16)]
            didx_v[pl.ds(j * 16, 16)] = sd_v[pl.ds(_C + j * 16, 16)]
        for g in range(_G):
            si8 = sidx_v[pl.ds(g * 16, 16)] * 8
            di8 = didx_v[pl.ds(g * 16, 16)] * 8
            ee = []
            for h in range(_H):
                el = plsc.load_gather(tab_v, [si8 + h])
                er = plsc.load_gather(tab_v, [di8 + (4 + h)])
                ee.append(jnp.exp(_leaky(el + er)))
            for i in range(16):
                ee_v[g * 16 + i] = (ee[0][i] * basis[0] + ee[1][i] * basis[1]
                                    + ee[2][i] * basis[2] + basis[3])
        pltpu.sync_copy(ee_v, acc_sh.at[didx_v], add=True)
        return carry
    lax.fori_loop(0, _NCHUNK, _chunk, 0)
    plsc.subcore_barrier()
    pltpu.sync_copy(acc_sh.at[pl.ds(s * _RPT, _RPT)],
                    sacc_hbm.at[c, pl.ds(s * _RPT, _RPT)])


_edge1 = pl.kernel(
    _edge1_body,
    out_type=[jax.ShapeDtypeStruct((2, _NP, 16), jnp.float32)],
    mesh=plsc.VectorSubcoreMesh(core_axis_name="c", subcore_axis_name="s"),
    compiler_params=pltpu.CompilerParams(needs_layout_passes=False, use_tc_tiling_on_sc=False),
    scratch_types=[
        pltpu.VMEM((_N * 8,), jnp.float32),
        pltpu.VMEM((2 * _C,), jnp.int32),
        pltpu.VMEM((_C,), jnp.int32),
        pltpu.VMEM((_C,), jnp.int32),
        pltpu.VMEM((_C, 16), jnp.float32),
        pltpu.VMEM((_RPT, 16), jnp.float32),
        pltpu.VMEM_SHARED((_NP, 16), jnp.float32),
    ],
)


def _tab2_body(sacc_ref, tab2_ref):
    t = sacc_ref[0] + sacc_ref[1]
    x = t[:, 0:4]
    col = lax.broadcasted_iota(jnp.int32, (_BLKP, 4), 1)
    sinv = 1.0 / jnp.maximum(x, 1e-16)
    dn = lax.rsqrt(jnp.maximum(x, 1.0))
    tab2_ref[...] = jnp.where(col < 3, sinv, dn)


def _tab2(sacc):
    return pl.pallas_call(
        _tab2_body,
        grid=(_NP // _BLKP,),
        in_specs=[pl.BlockSpec((2, _BLKP, 16), lambda i: (0, i, 0))],
        out_specs=pl.BlockSpec((_BLKP, 4), lambda i: (i, 0)),
        out_shape=jax.ShapeDtypeStruct((_NP, 4), jnp.float32),
    )(sacc)


def _edge2_body(ebuf_hbm, tab1_hbm, tab2_hbm, a_hbm,
                tab_v, tab2_v, sd_v, sidx_v, didx_v, a_v):
    c = lax.axis_index("c")
    s = lax.axis_index("s")
    wid = s * 2 + c
    pltpu.sync_copy(tab1_hbm, tab_v)
    pltpu.sync_copy(tab2_hbm, tab2_v)

    lane = lax.iota(jnp.int32, 16)
    basis = [jnp.where(lane == k, 1.0, 0.0) for k in range(16)]

    def _chunk(t, carry):
        base = wid * _EPW + t * _C
        pltpu.sync_copy(ebuf_hbm.at[wid * _NCHUNK + t], sd_v)
        for j in range(_C // 16):
            sidx_v[pl.ds(j * 16, 16)] = sd_v[pl.ds(j * 16, 16)]
            didx_v[pl.ds(j * 16, 16)] = sd_v[pl.ds(_C + j * 16, 16)]
        for g in range(_G):
            si = sidx_v[pl.ds(g * 16, 16)]
            di = didx_v[pl.ds(g * 16, 16)]
            si8 = si * 8
            di8 = di * 8
            di4 = di * 4
            dd = (plsc.load_gather(tab2_v, [si * 4 + 3])
                  * plsc.load_gather(tab2_v, [di4 + 3]))
            av = []
            for h in range(_H):
                el = plsc.load_gather(tab_v, [si8 + h])
                er = plsc.load_gather(tab_v, [di8 + (4 + h)])
                ee = jnp.exp(_leaky(el + er))
                sinv = plsc.load_gather(tab2_v, [di4 + h])
                av.append(ee * sinv * dd)
            for q in range(4):
                row = jnp.zeros((16,), jnp.float32)
                for j in range(4):
                    for h in range(_H):
                        row = row + av[h][4 * q + j] * basis[4 * j + h]
                a_v[pl.ds((g * 4 + q) * 16, 16)] = row
        pltpu.sync_copy(a_v, a_hbm.at[pl.ds(base * 4, _C * 4)])
        return carry
    lax.fori_loop(0, _NCHUNK, _chunk, 0)


_edge2 = pl.kernel(
    _edge2_body,
    out_type=[jax.ShapeDtypeStruct((_E * 4,), jnp.float32)],
    mesh=plsc.VectorSubcoreMesh(core_axis_name="c", subcore_axis_name="s"),
    compiler_params=pltpu.CompilerParams(needs_layout_passes=False, use_tc_tiling_on_sc=False),
    scratch_types=[
        pltpu.VMEM((_N * 8,), jnp.float32),
        pltpu.VMEM((_NP * 4,), jnp.float32),
        pltpu.VMEM((2 * _C,), jnp.int32),
        pltpu.VMEM((_C,), jnp.int32),
        pltpu.VMEM((_C,), jnp.int32),
        pltpu.VMEM((_C * 4,), jnp.float32),
    ],
)


def _hop_body(ebuf_hbm, a_hbm, h_hbm, out_hbm,
              sd0_v, sd1_v, sd2_v, sidx0_v, sidx1_v, sidx2_v,
              didx0_v, didx1_v, didx2_v, a0_v, a1_v, a2_v,
              rows0_v, rows1_v, rows2_v, zbuf_v, acc_sh,
              gsem0, gsem1, gsem2, asem0, asem1, asem2,
              ssem0, ssem1, ssem2, dsem0, dsem1, dsem2):
    c = lax.axis_index("c")
    s = lax.axis_index("s")
    lo = c * _NH

    z16 = jnp.zeros((16,), jnp.float32)
    lane = lax.iota(jnp.int32, 16)
    rows_b = (rows0_v, rows1_v, rows2_v)
    sd_b = (sd0_v, sd1_v, sd2_v)
    sidx_b = (sidx0_v, sidx1_v, sidx2_v)
    didx_b = (didx0_v, didx1_v, didx2_v)
    a_b = (a0_v, a1_v, a2_v)
    gsem_b = (gsem0, gsem1, gsem2)
    asem_b = (asem0, asem1, asem2)
    ssem_b = (ssem0, ssem1, ssem2)
    dsem_b = (dsem0, dsem1, dsem2)

    def _zrow(i, carry):
        for j in range(_HF // 16):
            zbuf_v[i, pl.ds(j * 16, 16)] = z16
        return carry
    lax.fori_loop(0, _ZR, _zrow, 0)
    for r in range(_RPTH // _ZR):
        pltpu.sync_copy(zbuf_v, acc_sh.at[pl.ds(s * _RPTH + r * _ZR, _ZR)])
    plsc.subcore_barrier()

    def _sd_issue(t, k):
        pltpu.async_copy(ebuf_hbm.at[s * _NCH2 + t], sd_b[k], dsem_b[k])

    def _issue(t, k):
        base = s * _EPT + t * _C
        pltpu.make_async_copy(ebuf_hbm.at[s * _NCH2 + t], sd_b[k],
                              dsem_b[k]).wait()
        for j in range(_C // 16):
            sidx_b[k][pl.ds(j * 16, 16)] = sd_b[k][pl.ds(j * 16, 16)]
            didx_b[k][pl.ds(j * 16, 16)] = sd_b[k][pl.ds(_C + j * 16, 16)]
        pltpu.async_copy(h_hbm.at[sidx_b[k]], rows_b[k], gsem_b[k])
        pltpu.async_copy(a_hbm.at[pl.ds(base * 4, _C * 4)], a_b[k],
                         asem_b[k])

    def _wait_issue(t, k):
        base = s * _EPT + t * _C
        pltpu.make_async_copy(h_hbm.at[sidx_b[k]], rows_b[k],
                              gsem_b[k]).wait()
        pltpu.make_async_copy(a_hbm.at[pl.ds(base * 4, _C * 4)], a_b[k],
                              asem_b[k]).wait()

    def _scale(k):
        def _scaleg(g, carry2):
            dv = didx_b[k][pl.ds(g * 16, 16)]
            dil = dv - lo
            ind = jnp.where((dil >= 0) & (dil < _NH), 1.0, 0.0)
            didx_b[k][pl.ds(g * 16, 16)] = jnp.minimum(
                jnp.maximum(dil, 0), _NH - 1)
            rows4 = (lane + g * 16) * 4
            av = [plsc.load_gather(a_b[k], [rows4 + h]) * ind
                  for h in range(_H)]
            for i in range(16):
                gi = g * 16 + i
                for h in range(_H):
                    coef = av[h][i]
                    for j in range(_F // 16):
                        sl = pl.ds(h * _F + j * 16, 16)
                        rows_b[k][gi, sl] = rows_b[k][gi, sl] * coef
            return carry2
        lax.fori_loop(0, _G, _scaleg, 0)

    def _drain_scatter(k):
        pltpu.make_async_copy(rows_b[k], acc_sh.at[didx_b[k]],
                              ssem_b[k]).wait()

    def _step(t, k, wait_sc, issue_next, sd_next):
        if sd_next:
            _sd_issue(t + 2, (k + 2) % 3)
        if issue_next:
            _issue(t + 1, (k + 1) % 3)
        _wait_issue(t, k)
        _scale(k)
        if wait_sc:
            _drain_scatter((k + 2) % 3)
        pltpu.async_copy(rows_b[k], acc_sh.at[didx_b[k]], ssem_b[k],
                         add=True)

    _sd_issue(0, 0)
    _sd_issue(1, 1)
    _issue(0, 0)
    _step(0, 0, False, True, True)
    _step(1, 1, True, True, True)

    def _trip(T, carry):
        t = 3 * T + 2
        _step(t, 2, True, True, True)
        _step(t + 1, 0, True, True, True)
        _step(t + 2, 1, True, True, True)
        return carry
    lax.fori_loop(0, (_NCH2 - 7) // 3, _trip, 0)
    _step(_NCH2 - 5, (_NCH2 - 5) % 3, True, True, True)
    _step(_NCH2 - 4, (_NCH2 - 4) % 3, True, True, True)
    _step(_NCH2 - 3, (_NCH2 - 3) % 3, True, True, True)
    _step(_NCH2 - 2, (_NCH2 - 2) % 3, True, True, False)
    _step(_NCH2 - 1, (_NCH2 - 1) % 3, True, False, False)
    _drain_scatter((_NCH2 - 1) % 3)
    plsc.subcore_barrier()
    pltpu.sync_copy(acc_sh.at[pl.ds(s * _RPTH, _RPTH)],
                    out_hbm.at[pl.ds(lo + s * _RPTH, _RPTH)])


_hop = pl.kernel(
    _hop_body,
    out_type=[jax.ShapeDtypeStruct((_NP, _HF), jnp.float32)],
    mesh=plsc.VectorSubcoreMesh(core_axis_name="c", subcore_axis_name="s"),
    compiler_params=pltpu.CompilerParams(needs_layout_passes=False, use_tc_tiling_on_sc=False),
    scratch_types=(
        [pltpu.VMEM((2 * _C,), jnp.int32)] * 3
        + [pltpu.VMEM((_C,), jnp.int32)] * 6
        + [pltpu.VMEM((_C * 4,), jnp.float32)] * 3
        + [pltpu.VMEM((_C, _HF), jnp.float32)] * 3
        + [pltpu.VMEM((_ZR, _HF), jnp.float32),
           pltpu.VMEM_SHARED((_NH, _HF), jnp.float32)]
        + [pltpu.SemaphoreType.DMA] * 12
    ),
)


def _final_body(ft_ref, h1_ref, h2_ref, h3_ref, pos_ref, hl_ref, hr_ref,
                out_ref):
    g0 = ft_ref[...] + pos_ref[0:1, :]
    g1 = h1_ref[...] + pos_ref[1:2, :]
    g2 = h2_ref[...] + pos_ref[2:3, :]
    g3 = h3_ref[...] + pos_ref[3:4, :]
    gs = (g0, g1, g2, g3)
    al = jnp.dot(g0, hl_ref[...], preferred_element_type=jnp.float32)
    ah = [_leaky(jnp.dot(gk, hr_ref[...], preferred_element_type=jnp.float32)
                 + al) for gk in gs]
    m = jnp.maximum(jnp.maximum(ah[0], ah[1]), jnp.maximum(ah[2], ah[3]))
    ek = [jnp.exp(t - m) for t in ah]
    ssum = ek[0] + ek[1] + ek[2] + ek[3]
    wk = [t / ssum for t in ek]
    for h in range(_H):
        acc = gs[0][:, h * _F:(h + 1) * _F] * wk[0][:, h:h + 1]
        for k in range(1, _K + 1):
            acc = acc + gs[k][:, h * _F:(h + 1) * _F] * wk[k][:, h:h + 1]
        out_ref[:, h * _F:(h + 1) * _F] = acc


def _final(ft, h1, h2, h3, pos, hl8, hr8):
    return pl.pallas_call(
        _final_body,
        grid=(_N // _BLK,),
        in_specs=[pl.BlockSpec((_BLK, _HF), lambda i: (i, 0)),
                  pl.BlockSpec((_BLK, _HF), lambda i: (i, 0)),
                  pl.BlockSpec((_BLK, _HF), lambda i: (i, 0)),
                  pl.BlockSpec((_BLK, _HF), lambda i: (i, 0)),
                  pl.BlockSpec((_K + 1, _HF), lambda i: (0, 0)),
                  pl.BlockSpec((_HF, 8), lambda i: (0, 0)),
                  pl.BlockSpec((_HF, 8), lambda i: (0, 0))],
        out_specs=pl.BlockSpec((_BLK, _HF), lambda i: (i, 0)),
        out_shape=jax.ShapeDtypeStruct((_N, _HF), jnp.float32),
    )(ft, h1, h2, h3, pos, hl8, hr8)


def _pack8(w, col0):
    w = w.reshape(_H, _F)
    m = jnp.zeros((_HF, 8), jnp.float32)
    for h in range(_H):
        m = m.at[h * _F:(h + 1) * _F, col0 + h].set(w[h])
    return m


def kernel(feat, edge_index, W_fc, attn_l, attn_r, position_emb,
           hop_attn_l, hop_attn_r):
    src = edge_index[0].astype(jnp.int32)
    dst = edge_index[1].astype(jnp.int32)
    wt = W_fc.T
    a8 = _pack8(attn_l, 0) + _pack8(attn_r, 4)
    hl8 = _pack8(hop_attn_l, 0)
    hr8 = _pack8(hop_attn_r, 0)
    pos = position_emb.reshape(_K + 1, _HF)

    ebuf = jnp.concatenate([src.reshape(_E // _C, _C),
                            dst.reshape(_E // _C, _C)], axis=1)
    ft, tab1 = _fc(feat, wt, a8)
    tab1f = tab1.reshape(_N * 8)
    (sacc,) = _edge1(ebuf, tab1f)
    tab2f = _tab2(sacc).reshape(_NP * 4)
    (a,) = _edge2(ebuf, tab1f, tab2f)
    (h1,) = _hop(ebuf, a, ft)
    (h2,) = _hop(ebuf, a, h1)
    (h3,) = _hop(ebuf, a, h2)
    rst = _final(ft, h1, h2, h3, pos, hl8, hr8)
    return rst.reshape(_N, _H, _F)


# speedup vs baseline: 1.0291x; 1.0291x over previous
import jax
import jax.numpy as jnp
from jax import lax
from jax.experimental import pallas as pl
from jax.experimental.pallas import tpu as pltpu
from jax.experimental.pallas import tpu_sc as plsc

_N = 10000
_E = 320000
_IN = 128
_H = 3
_F = 64
_HF = _H * _F
_K = 3
_NEG = 0.2

_NW = 32
_EPW = _E // _NW
_C = 80
_NCHUNK = _EPW // _C
_G = _C // 16
_NP = 10240
_RPT = _NP // 16
_NH = _NP // 2
_RPTH = _NH // 16
_EPT = _E // 16
_NCH2 = _EPT // _C
_ZR = 32

_BLK = 1000
_BLKP = 1024


def _leaky(x):
    return jnp.where(x >= 0, x, _NEG * x)


def _fc_body(feat_ref, wt_ref, a8_ref, ft_ref, tab1_ref):
    f = jnp.dot(feat_ref[...], wt_ref[...], preferred_element_type=jnp.float32)
    ft_ref[...] = f
    tab1_ref[...] = jnp.dot(f, a8_ref[...], preferred_element_type=jnp.float32)


def _fc(feat, wt, a8):
    return pl.pallas_call(
        _fc_body,
        grid=(_N // _BLK,),
        in_specs=[pl.BlockSpec((_BLK, _IN), lambda i: (i, 0)),
                  pl.BlockSpec((_IN, _HF), lambda i: (0, 0)),
                  pl.BlockSpec((_HF, 8), lambda i: (0, 0))],
        out_specs=[pl.BlockSpec((_BLK, _HF), lambda i: (i, 0)),
                   pl.BlockSpec((_BLK, 8), lambda i: (i, 0))],
        out_shape=[jax.ShapeDtypeStruct((_N, _HF), jnp.float32),
                   jax.ShapeDtypeStruct((_N, 8), jnp.float32)],
    )(feat, wt, a8)


def _edge1_body(ebuf_hbm, tab1_hbm, sacc_hbm,
                tab_v, sd_v, sidx_v, didx_v, ee_v, zbuf_v, acc_sh):
    c = lax.axis_index("c")
    s = lax.axis_index("s")
    wid = s * 2 + c
    pltpu.sync_copy(tab1_hbm, tab_v)

    z16 = jnp.zeros((16,), jnp.float32)
    lane = lax.iota(jnp.int32, 16)
    basis = [jnp.where(lane == k, 1.0, 0.0) for k in range(4)]

    def _zrow(i, carry):
        zbuf_v[i] = z16
        return carry
    lax.fori_loop(0, _RPT, _zrow, 0)
    pltpu.sync_copy(zbuf_v, acc_sh.at[pl.ds(s * _RPT, _RPT)])
    plsc.subcore_barrier()

    def _chunk(t, carry):
        pltpu.sync_copy(ebuf_hbm.at[wid * _NCHUNK + t], sd_v)
        for j in range(_C // 16):
            sidx_v[pl.ds(j * 16, 16)] = sd_v[pl.ds(j * 16, 16)]
            didx_v[pl.ds(j * 16, 16)] = sd_v[pl.ds(_C + j * 16, 16)]
        for g in range(_G):
            si8 = sidx_v[pl.ds(g * 16, 16)] * 8
            di8 = didx_v[pl.ds(g * 16, 16)] * 8
            ee = []
            for h in range(_H):
                el = plsc.load_gather(tab_v, [si8 + h])
                er = plsc.load_gather(tab_v, [di8 + (4 + h)])
                ee.append(jnp.exp(_leaky(el + er)))
            for i in range(16):
                ee_v[g * 16 + i] = (ee[0][i] * basis[0] + ee[1][i] * basis[1]
                                    + ee[2][i] * basis[2] + basis[3])
        pltpu.sync_copy(ee_v, acc_sh.at[didx_v], add=True)
        return carry
    lax.fori_loop(0, _NCHUNK, _chunk, 0)
    plsc.subcore_barrier()
    pltpu.sync_copy(acc_sh.at[pl.ds(s * _RPT, _RPT)],
                    sacc_hbm.at[c, pl.ds(s * _RPT, _RPT)])


_edge1 = pl.kernel(
    _edge1_body,
    out_type=[jax.ShapeDtypeStruct((2, _NP, 16), jnp.float32)],
    mesh=plsc.VectorSubcoreMesh(core_axis_name="c", subcore_axis_name="s"),
    compiler_params=pltpu.CompilerParams(needs_layout_passes=False, use_tc_tiling_on_sc=False),
    scratch_types=[
        pltpu.VMEM((_N * 8,), jnp.float32),
        pltpu.VMEM((2 * _C,), jnp.int32),
        pltpu.VMEM((_C,), jnp.int32),
        pltpu.VMEM((_C,), jnp.int32),
        pltpu.VMEM((_C, 16), jnp.float32),
        pltpu.VMEM((_RPT, 16), jnp.float32),
        pltpu.VMEM_SHARED((_NP, 16), jnp.float32),
    ],
)


def _tab2_body(sacc_ref, tab2_ref):
    t = sacc_ref[0] + sacc_ref[1]
    x = t[:, 0:4]
    col = lax.broadcasted_iota(jnp.int32, (_BLKP, 4), 1)
    sinv = 1.0 / jnp.maximum(x, 1e-16)
    dn = lax.rsqrt(jnp.maximum(x, 1.0))
    tab2_ref[...] = jnp.where(col < 3, sinv, dn)


def _tab2(sacc):
    return pl.pallas_call(
        _tab2_body,
        grid=(_NP // _BLKP,),
        in_specs=[pl.BlockSpec((2, _BLKP, 16), lambda i: (0, i, 0))],
        out_specs=pl.BlockSpec((_BLKP, 4), lambda i: (i, 0)),
        out_shape=jax.ShapeDtypeStruct((_NP, 4), jnp.float32),
    )(sacc)


def _edge2_body(ebuf_hbm, tab1_hbm, tab2_hbm, a_hbm,
                tab_v, tab2_v, sd_v, sidx_v, didx_v, a_v):
    c = lax.axis_index("c")
    s = lax.axis_index("s")
    wid = s * 2 + c
    pltpu.sync_copy(tab1_hbm, tab_v)
    pltpu.sync_copy(tab2_hbm, tab2_v)

    lane = lax.iota(jnp.int32, 16)
    basis = [jnp.where(lane == k, 1.0, 0.0) for k in range(16)]

    def _chunk(t, carry):
        base = wid * _EPW + t * _C
        pltpu.sync_copy(ebuf_hbm.at[wid * _NCHUNK + t], sd_v)
        for j in range(_C // 16):
            sidx_v[pl.ds(j * 16, 16)] = sd_v[pl.ds(j * 16, 16)]
            didx_v[pl.ds(j * 16, 16)] = sd_v[pl.ds(_C + j * 16, 16)]
        for g in range(_G):
            si = sidx_v[pl.ds(g * 16, 16)]
            di = didx_v[pl.ds(g * 16, 16)]
            si8 = si * 8
            di8 = di * 8
            di4 = di * 4
            dd = (plsc.load_gather(tab2_v, [si * 4 + 3])
                  * plsc.load_gather(tab2_v, [di4 + 3]))
            av = []
            for h in range(_H):
                el = plsc.load_gather(tab_v, [si8 + h])
                er = plsc.load_gather(tab_v, [di8 + (4 + h)])
                ee = jnp.exp(_leaky(el + er))
                sinv = plsc.load_gather(tab2_v, [di4 + h])
                av.append(ee * sinv * dd)
            for q in range(4):
                row = jnp.zeros((16,), jnp.float32)
                for j in range(4):
                    for h in range(_H):
                        row = row + av[h][4 * q + j] * basis[4 * j + h]
                a_v[pl.ds((g * 4 + q) * 16, 16)] = row
        pltpu.sync_copy(a_v, a_hbm.at[pl.ds(base * 4, _C * 4)])
        return carry
    lax.fori_loop(0, _NCHUNK, _chunk, 0)


_edge2 = pl.kernel(
    _edge2_body,
    out_type=[jax.ShapeDtypeStruct((_E * 4,), jnp.float32)],
    mesh=plsc.VectorSubcoreMesh(core_axis_name="c", subcore_axis_name="s"),
    compiler_params=pltpu.CompilerParams(needs_layout_passes=False, use_tc_tiling_on_sc=False),
    scratch_types=[
        pltpu.VMEM((_N * 8,), jnp.float32),
        pltpu.VMEM((_NP * 4,), jnp.float32),
        pltpu.VMEM((2 * _C,), jnp.int32),
        pltpu.VMEM((_C,), jnp.int32),
        pltpu.VMEM((_C,), jnp.int32),
        pltpu.VMEM((_C * 4,), jnp.float32),
    ],
)


def _hop_body(ebuf_hbm, a_hbm, h_hbm, out_hbm,
              sd0_v, sd1_v, sd2_v, sidx0_v, sidx1_v, sidx2_v,
              didx0_v, didx1_v, didx2_v, a0_v, a1_v, a2_v,
              rows0_v, rows1_v, rows2_v, zbuf_v, acc_sh,
              gsem0, gsem1, gsem2, asem0, asem1, asem2,
              ssem0, ssem1, ssem2):
    c = lax.axis_index("c")
    s = lax.axis_index("s")
    lo = c * _NH

    z16 = jnp.zeros((16,), jnp.float32)
    lane = lax.iota(jnp.int32, 16)
    rows_b = (rows0_v, rows1_v, rows2_v)
    sd_b = (sd0_v, sd1_v, sd2_v)
    sidx_b = (sidx0_v, sidx1_v, sidx2_v)
    didx_b = (didx0_v, didx1_v, didx2_v)
    a_b = (a0_v, a1_v, a2_v)
    gsem_b = (gsem0, gsem1, gsem2)
    asem_b = (asem0, asem1, asem2)
    ssem_b = (ssem0, ssem1, ssem2)

    def _zrow(i, carry):
        for j in range(_HF // 16):
            zbuf_v[i, pl.ds(j * 16, 16)] = z16
        return carry
    lax.fori_loop(0, _ZR, _zrow, 0)
    for r in range(_RPTH // _ZR):
        pltpu.sync_copy(zbuf_v, acc_sh.at[pl.ds(s * _RPTH + r * _ZR, _ZR)])
    plsc.subcore_barrier()

    def _issue(t, k):
        base = s * _EPT + t * _C
        pltpu.sync_copy(ebuf_hbm.at[s * _NCH2 + t], sd_b[k])
        for j in range(_C // 16):
            sidx_b[k][pl.ds(j * 16, 16)] = sd_b[k][pl.ds(j * 16, 16)]
            didx_b[k][pl.ds(j * 16, 16)] = sd_b[k][pl.ds(_C + j * 16, 16)]
        pltpu.async_copy(h_hbm.at[sidx_b[k]], rows_b[k], gsem_b[k])
        pltpu.async_copy(a_hbm.at[pl.ds(base * 4, _C * 4)], a_b[k],
                         asem_b[k])

    def _wait_issue(t, k):
        base = s * _EPT + t * _C
        pltpu.make_async_copy(h_hbm.at[sidx_b[k]], rows_b[k],
                              gsem_b[k]).wait()
        pltpu.make_async_copy(a_hbm.at[pl.ds(base * 4, _C * 4)], a_b[k],
                              asem_b[k]).wait()

    def _scale(k):
        def _scaleg(g, carry2):
            dv = didx_b[k][pl.ds(g * 16, 16)]
            dil = dv - lo
            ind = jnp.where((dil >= 0) & (dil < _NH), 1.0, 0.0)
            didx_b[k][pl.ds(g * 16, 16)] = jnp.minimum(
                jnp.maximum(dil, 0), _NH - 1)
            rows4 = (lane + g * 16) * 4
            av = [plsc.load_gather(a_b[k], [rows4 + h]) * ind
                  for h in range(_H)]
            for i in range(16):
                gi = g * 16 + i
                for h in range(_H):
                    coef = av[h][i]
                    for j in range(_F // 16):
                        sl = pl.ds(h * _F + j * 16, 16)
                        rows_b[k][gi, sl] = rows_b[k][gi, sl] * coef
            return carry2
        lax.fori_loop(0, _G, _scaleg, 0)

    def _drain_scatter(k):
        pltpu.make_async_copy(rows_b[k], acc_sh.at[didx_b[k]],
                              ssem_b[k]).wait()

    def _step(t, k, wait_sc, issue_next):
        _wait_issue(t, k)
        _scale(k)
        if wait_sc:
            _drain_scatter((k + 2) % 3)
        if issue_next:
            _issue(t + 2, (k + 2) % 3)
        pltpu.async_copy(rows_b[k], acc_sh.at[didx_b[k]], ssem_b[k],
                         add=True)

    _issue(0, 0)
    _issue(1, 1)
    _step(0, 0, False, True)
    _step(1, 1, True, True)

    def _trip(T, carry):
        t = 3 * T + 2
        _step(t, 2, True, True)
        _step(t + 1, 0, True, True)
        _step(t + 2, 1, True, True)
        return carry
    lax.fori_loop(0, (_NCH2 - 4) // 3, _trip, 0)
    _step(_NCH2 - 2, (_NCH2 - 2) % 3, True, False)
    _step(_NCH2 - 1, (_NCH2 - 1) % 3, True, False)
    _drain_scatter((_NCH2 - 1) % 3)
    plsc.subcore_barrier()
    pltpu.sync_copy(acc_sh.at[pl.ds(s * _RPTH, _RPTH)],
                    out_hbm.at[pl.ds(lo + s * _RPTH, _RPTH)])


_hop = pl.kernel(
    _hop_body,
    out_type=[jax.ShapeDtypeStruct((_NP, _HF), jnp.float32)],
    mesh=plsc.VectorSubcoreMesh(core_axis_name="c", subcore_axis_name="s"),
    compiler_params=pltpu.CompilerParams(needs_layout_passes=False, use_tc_tiling_on_sc=False),
    scratch_types=(
        [pltpu.VMEM((2 * _C,), jnp.int32)] * 3
        + [pltpu.VMEM((_C,), jnp.int32)] * 6
        + [pltpu.VMEM((_C * 4,), jnp.float32)] * 3
        + [pltpu.VMEM((_C, _HF), jnp.float32)] * 3
        + [pltpu.VMEM((_ZR, _HF), jnp.float32),
           pltpu.VMEM_SHARED((_NH, _HF), jnp.float32)]
        + [pltpu.SemaphoreType.DMA] * 9
    ),
)


def _final_body(ft_ref, h1_ref, h2_ref, h3_ref, pos_ref, hl_ref, hr_ref,
                out_ref):
    g0 = ft_ref[...] + pos_ref[0:1, :]
    g1 = h1_ref[...] + pos_ref[1:2, :]
    g2 = h2_ref[...] + pos_ref[2:3, :]
    g3 = h3_ref[...] + pos_ref[3:4, :]
    gs = (g0, g1, g2, g3)
    al = jnp.dot(g0, hl_ref[...], preferred_element_type=jnp.float32)
    ah = [_leaky(jnp.dot(gk, hr_ref[...], preferred_element_type=jnp.float32)
                 + al) for gk in gs]
    m = jnp.maximum(jnp.maximum(ah[0], ah[1]), jnp.maximum(ah[2], ah[3]))
    ek = [jnp.exp(t - m) for t in ah]
    ssum = ek[0] + ek[1] + ek[2] + ek[3]
    wk = [t / ssum for t in ek]
    for h in range(_H):
        acc = gs[0][:, h * _F:(h + 1) * _F] * wk[0][:, h:h + 1]
        for k in range(1, _K + 1):
            acc = acc + gs[k][:, h * _F:(h + 1) * _F] * wk[k][:, h:h + 1]
        out_ref[:, h * _F:(h + 1) * _F] = acc


def _final(ft, h1, h2, h3, pos, hl8, hr8):
    return pl.pallas_call(
        _final_body,
        grid=(_N // _BLK,),
        in_specs=[pl.BlockSpec((_BLK, _HF), lambda i: (i, 0)),
                  pl.BlockSpec((_BLK, _HF), lambda i: (i, 0)),
                  pl.BlockSpec((_BLK, _HF), lambda i: (i, 0)),
                  pl.BlockSpec((_BLK, _HF), lambda i: (i, 0)),
                  pl.BlockSpec((_K + 1, _HF), lambda i: (0, 0)),
                  pl.BlockSpec((_HF, 8), lambda i: (0, 0)),
                  pl.BlockSpec((_HF, 8), lambda i: (0, 0))],
        out_specs=pl.BlockSpec((_BLK, _HF), lambda i: (i, 0)),
        out_shape=jax.ShapeDtypeStruct((_N, _HF), jnp.float32),
    )(ft, h1, h2, h3, pos, hl8, hr8)


def _pack8(w, col0):
    w = w.reshape(_H, _F)
    m = jnp.zeros((_HF, 8), jnp.float32)
    for h in range(_H):
        m = m.at[h * _F:(h + 1) * _F, col0 + h].set(w[h])
    return m


def kernel(feat, edge_index, W_fc, attn_l, attn_r, position_emb,
           hop_attn_l, hop_attn_r):
    src = edge_index[0].astype(jnp.int32)
    dst = edge_index[1].astype(jnp.int32)
    wt = W_fc.T
    a8 = _pack8(attn_l, 0) + _pack8(attn_r, 4)
    hl8 = _pack8(hop_attn_l, 0)
    hr8 = _pack8(hop_attn_r, 0)
    pos = position_emb.reshape(_K + 1, _HF)

    ebuf = jnp.concatenate([src.reshape(_E // _C, _C),
                            dst.reshape(_E // _C, _C)], axis=1)
    ft, tab1 = _fc(feat, wt, a8)
    tab1f = tab1.reshape(_N * 8)
    (sacc,) = _edge1(ebuf, tab1f)
    tab2f = _tab2(sacc).reshape(_NP * 4)
    (a,) = _edge2(ebuf, tab1f, tab2f)
    (h1,) = _hop(ebuf, a, ft)
    (h2,) = _hop(ebuf, a, h1)
    (h3,) = _hop(ebuf, a, h2)
    rst = _final(ft, h1, h2, h3, pos, hl8, hr8)
    return rst.reshape(_N, _H, _F)
